# baseline (speedup 1.0000x reference)
import functools

import jax
import jax.numpy as jnp
from jax import lax
from jax.experimental import pallas as pl
from jax.experimental.pallas import tpu as pltpu
from jax.experimental.pallas import tpu_sc as plsc

VOCAB_N = 100000
EMB_D = 128
BATCH_B = 1024

TILE_V = 2000


@functools.cache
def _make_sc_gather():
    info = plsc.get_sparse_core_info()
    ns = info.num_subcores
    nc = 1
    nw = nc * ns
    b_per_w = BATCH_B // nw
    mesh = plsc.VectorSubcoreMesh(
        core_axis_name="c", subcore_axis_name="s", num_cores=1
    )

    @functools.partial(
        pl.kernel,
        mesh=mesh,
        out_type=jax.ShapeDtypeStruct((BATCH_B, EMB_D), jnp.float32),
        scratch_types=[
            pltpu.VMEM((b_per_w,), jnp.int32),
            pltpu.VMEM((b_per_w, EMB_D), jnp.float32),
            pltpu.SemaphoreType.DMA,
        ],
    )
    def gather_kernel(table_hbm, idx_hbm, out_hbm, idx_v, rows_v, sem):
        wid = lax.axis_index("s") * nc + lax.axis_index("c")
        base = wid * b_per_w
        pltpu.sync_copy(idx_hbm.at[pl.ds(base, b_per_w)], idx_v)
        pltpu.async_copy(table_hbm.at[idx_v], rows_v, sem).wait()
        pltpu.sync_copy(rows_v, out_hbm.at[pl.ds(base, b_per_w)])

    return gather_kernel


def _mm_body(wt_ref, x_ref, o_ref):
    wb = wt_ref[...].astype(jnp.bfloat16)
    xb = x_ref[...].astype(jnp.bfloat16)
    o_ref[...] = lax.dot_general(
        wb,
        xb,
        dimension_numbers=(((1,), (1,)), ((), ())),
        preferred_element_type=jnp.float32,
    )


def _projection_t(wt, x):
    nt = pl.cdiv(VOCAB_N, TILE_V)
    return pl.pallas_call(
        _mm_body,
        grid=(nt,),
        in_specs=[
            pl.BlockSpec((TILE_V, EMB_D), lambda j: (j, 0)),
            pl.BlockSpec((BATCH_B, EMB_D), lambda j: (0, 0)),
        ],
        out_specs=pl.BlockSpec((TILE_V, BATCH_B), lambda j: (j, 0)),
        out_shape=jax.ShapeDtypeStruct((VOCAB_N, BATCH_B), jnp.float32),
    )(wt, x)


def kernel(input_ids, emb_table, W, b):
    x = _make_sc_gather()(emb_table, input_ids.astype(jnp.int32))
    out_t = _projection_t(W.T, x)
    out_t = lax.cond(
        jnp.any(b != 0.0),
        lambda o: o + b[:, None],
        lambda o: o,
        out_t,
    )
    return out_t.T


# speedup vs baseline: 1.0195x; 1.0195x over previous
import functools

import jax
import jax.numpy as jnp
from jax import lax
from jax.experimental import pallas as pl
from jax.experimental.pallas import tpu as pltpu
from jax.experimental.pallas import tpu_sc as plsc

VOCAB_N = 100000
EMB_D = 128
BATCH_B = 1024

TILE_V = 5000


@functools.cache
def _make_sc_gather():
    info = plsc.get_sparse_core_info()
    ns = info.num_subcores
    nc = 1
    nw = nc * ns
    b_per_w = BATCH_B // nw
    mesh = plsc.VectorSubcoreMesh(
        core_axis_name="c", subcore_axis_name="s", num_cores=1
    )

    @functools.partial(
        pl.kernel,
        mesh=mesh,
        out_type=jax.ShapeDtypeStruct((BATCH_B, EMB_D), jnp.float32),
        scratch_types=[
            pltpu.VMEM((b_per_w,), jnp.int32),
            pltpu.VMEM((b_per_w, EMB_D), jnp.float32),
            pltpu.SemaphoreType.DMA,
        ],
    )
    def gather_kernel(table_hbm, idx_hbm, out_hbm, idx_v, rows_v, sem):
        wid = lax.axis_index("s") * nc + lax.axis_index("c")
        base = wid * b_per_w
        pltpu.sync_copy(idx_hbm.at[pl.ds(base, b_per_w)], idx_v)
        pltpu.async_copy(table_hbm.at[idx_v], rows_v, sem).wait()
        pltpu.sync_copy(rows_v, out_hbm.at[pl.ds(base, b_per_w)])

    return gather_kernel


def _mm_body(wt_ref, x_ref, o_ref):
    wb = wt_ref[...].astype(jnp.bfloat16)
    xb = x_ref[...].astype(jnp.bfloat16)
    o_ref[...] = lax.dot_general(
        wb,
        xb,
        dimension_numbers=(((1,), (1,)), ((), ())),
        preferred_element_type=jnp.float32,
    )


def _projection_t(wt, x):
    nt = pl.cdiv(VOCAB_N, TILE_V)
    return pl.pallas_call(
        _mm_body,
        grid=(nt,),
        in_specs=[
            pl.BlockSpec((TILE_V, EMB_D), lambda j: (j, 0)),
            pl.BlockSpec((BATCH_B, EMB_D), lambda j: (0, 0)),
        ],
        out_specs=pl.BlockSpec((TILE_V, BATCH_B), lambda j: (j, 0)),
        out_shape=jax.ShapeDtypeStruct((VOCAB_N, BATCH_B), jnp.float32),
    )(wt, x)


def kernel(input_ids, emb_table, W, b):
    x = _make_sc_gather()(emb_table, input_ids.astype(jnp.int32))
    out_t = _projection_t(W.T, x)
    out_t = lax.cond(
        jnp.any(b != 0.0),
        lambda o: o + b[:, None],
        lambda o: o,
        out_t,
    )
    return out_t.T
